# baseline (speedup 1.0000x reference)
import jax
import jax.numpy as jnp
from jax import lax
from jax.experimental import pallas as pl
from jax.experimental.pallas import tpu as pltpu
from jax.experimental.pallas import tpu_sc as plsc

VOCAB = 1000000
EMB_DIM = 64
NC = 2
NS = 16
NW = NC * NS

B_I = 16384
B_J = 50
LANE = 128
N_IT = B_I // LANE
IT_PER_W = N_IT // NW
UNITS = IT_PER_W * B_J
NBUF = 3


def _emb_body(x_hbm, tpk_hbm, out_hbm, ibuf, sidx, par, gbs, rbs, gsems,
              ssems):
    wid = lax.axis_index("s") * NC + lax.axis_index("c")
    iota = lax.iota(jnp.int32, 16)

    def p1_body(itl, carry):
        col = (wid * IT_PER_W + itl) * LANE

        def jt_body(jt, c2):
            pltpu.sync_copy(
                x_hbm.at[pl.ds(jt * 8, 8), pl.ds(col, LANE)], ibuf)
            nrows = lax.min(B_J - jt * 8, 8)

            def jr_body(jr, c3):
                u = itl * B_J + jt * 8 + jr
                for c in range(8):
                    v = ibuf[jr, pl.ds(c * 16, 16)]
                    sidx[u, pl.ds(c * 16, 16)] = lax.shift_right_logical(v, 1)
                    par[u, pl.ds(c * 16, 16)] = lax.shift_left(
                        lax.bitwise_and(v, 1), 6)
                return c3

            lax.fori_loop(0, nrows, jr_body, 0)
            return c2

        lax.fori_loop(0, (B_J + 7) // 8, jt_body, 0)
        return carry

    lax.fori_loop(0, IT_PER_W, p1_body, 0)

    def fire(u, b):
        pltpu.async_copy(tpk_hbm.at[sidx.at[u]], gbs[b], gsems[b])

    def proc(u, j, col, b):
        gb, rb, gsem, ssem = gbs[b], rbs[b], gsems[b], ssems[b]

        @pl.when(u >= NBUF)
        def _():
            pltpu.make_async_copy(
                rb, out_hbm.at[0, :, pl.ds(0, LANE)], ssem).wait()
        pltpu.make_async_copy(tpk_hbm.at[sidx.at[u]], gb, gsem).wait()
        for c in range(8):
            pvec = par[u, pl.ds(c * 16, 16)]
            rows_c = iota + (c * 16)

            @plsc.parallel_loop(0, EMB_DIM, unroll=16)
            def _(d):
                rb[d, pl.ds(c * 16, 16)] = plsc.load_gather(
                    gb, [rows_c, pvec + d])
        pltpu.async_copy(rb, out_hbm.at[j, :, pl.ds(col, LANE)], ssem)

    for b in range(NBUF):
        fire(b, b)

    def u_body(i, carry):
        j, itl = carry
        for q in range(NBUF):
            u = NBUF * i + q
            col = (wid * IT_PER_W + itl) * LANE
            proc(u, j, col, q)

            @pl.when(u + NBUF < UNITS)
            def _():
                fire(u + NBUF, q)
            jn = j + 1
            wrap = jn >= B_J
            j = lax.select(wrap, jnp.int32(0), jn)
            itl = itl + wrap.astype(jnp.int32)
        return (j, itl)

    lax.fori_loop(0, UNITS // NBUF, u_body,
                  (jnp.int32(0), jnp.int32(0)))

    rem = UNITS - (UNITS // NBUF) * NBUF
    for q in range(rem):
        u = UNITS - rem + q
        itl = u // B_J
        j = u % B_J
        col = (wid * IT_PER_W + itl) * LANE
        proc(u, j, col, u % NBUF)

    for b in range(NBUF):
        pltpu.make_async_copy(
            rbs[b], out_hbm.at[0, :, pl.ds(0, LANE)], ssems[b]).wait()


def kernel(x, table):
    x_t = x.T.astype(jnp.int32)
    tpk = jnp.reshape(table[:VOCAB], (VOCAB // 2, 128))

    mesh = plsc.VectorSubcoreMesh(core_axis_name="c", subcore_axis_name="s")
    out3 = pl.kernel(
        _emb_body,
        out_type=jax.ShapeDtypeStruct((B_J, EMB_DIM, B_I), jnp.float32),
        mesh=mesh,
        scratch_types=[
            pltpu.VMEM((8, LANE), jnp.int32),
            pltpu.VMEM((UNITS, LANE), jnp.int32),
            pltpu.VMEM((UNITS, LANE), jnp.int32),
            [pltpu.VMEM((LANE, LANE), jnp.float32) for _ in range(NBUF)],
            [pltpu.VMEM((EMB_DIM, LANE), jnp.float32) for _ in range(NBUF)],
            [pltpu.SemaphoreType.DMA for _ in range(NBUF)],
            [pltpu.SemaphoreType.DMA for _ in range(NBUF)],
        ],
        compiler_params=pltpu.CompilerParams(use_tc_tiling_on_sc=True,
                                             needs_layout_passes=False),
    )(x_t, tpk)
    return out3.transpose(2, 0, 1)


# speedup vs baseline: 1.0323x; 1.0323x over previous
import jax
import jax.numpy as jnp
from jax import lax
from jax.experimental import pallas as pl
from jax.experimental.pallas import tpu as pltpu
from jax.experimental.pallas import tpu_sc as plsc

VOCAB = 1000000
EMB_DIM = 64
NC = 2
NS = 16
NW = NC * NS

CHUNK = 128
K = 4
GROUP = K * CHUNK


def _emb_kernel_body(x_hbm, table_hbm, out_hbm, idx_v, buf0, buf1, gsem0,
                     gsem1, ssem0, ssem1):
    wid = lax.axis_index("s") * NC + lax.axis_index("c")
    n_idx_rows = idx_v.shape[0]
    n_groups = n_idx_rows // K
    base = wid * (n_idx_rows * CHUNK)

    pltpu.sync_copy(x_hbm.at[wid], idx_v)

    def fire_gathers(g, buf, gsem):
        for j in range(K):
            pltpu.async_copy(table_hbm.at[idx_v.at[g * K + j]],
                             buf.at[pl.ds(j * CHUNK, CHUNK)], gsem)

    def wait_gathers(buf, gsem):
        for j in range(K):
            pltpu.make_async_copy(out_hbm.at[pl.ds(0, CHUNK)],
                                  buf.at[pl.ds(j * CHUNK, CHUNK)], gsem).wait()

    def fire_store(g, buf, ssem):
        pltpu.async_copy(buf, out_hbm.at[pl.ds(base + g * GROUP, GROUP)], ssem)

    def wait_store(buf, ssem):
        pltpu.make_async_copy(buf, out_hbm.at[pl.ds(base, GROUP)], ssem).wait()

    fire_gathers(0, buf0, gsem0)
    fire_gathers(1, buf1, gsem1)
    wait_gathers(buf0, gsem0)
    fire_store(0, buf0, ssem0)

    def body(i, carry):
        g = 2 * i + 2
        wait_store(buf0, ssem0)
        fire_gathers(g, buf0, gsem0)
        wait_gathers(buf1, gsem1)
        fire_store(g - 1, buf1, ssem1)
        wait_store(buf1, ssem1)
        fire_gathers(g + 1, buf1, gsem1)
        wait_gathers(buf0, gsem0)
        fire_store(g, buf0, ssem0)
        return carry

    lax.fori_loop(0, (n_groups - 2) // 2, body, 0)

    wait_gathers(buf1, gsem1)
    fire_store(n_groups - 1, buf1, ssem1)
    wait_store(buf0, ssem0)
    wait_store(buf1, ssem1)


def kernel(x, table):
    B = x.shape[0] * x.shape[1]
    assert B % (NW * GROUP * 2) == 0
    n_idx_rows = B // (NW * CHUNK)
    x_flat = x.reshape(NW, n_idx_rows, CHUNK).astype(jnp.int32)

    mesh = plsc.VectorSubcoreMesh(core_axis_name="c", subcore_axis_name="s")
    out = pl.kernel(
        _emb_kernel_body,
        out_type=jax.ShapeDtypeStruct((B, EMB_DIM), jnp.float32),
        mesh=mesh,
        scratch_types=[
            pltpu.VMEM((n_idx_rows, CHUNK), jnp.int32),
            pltpu.VMEM((GROUP, EMB_DIM), jnp.float32),
            pltpu.VMEM((GROUP, EMB_DIM), jnp.float32),
            pltpu.SemaphoreType.DMA,
            pltpu.SemaphoreType.DMA,
            pltpu.SemaphoreType.DMA,
            pltpu.SemaphoreType.DMA,
        ],
        compiler_params=pltpu.CompilerParams(use_tc_tiling_on_sc=False),
    )(x_flat, table)
    return out.reshape(x.shape[0], x.shape[1], EMB_DIM)
